# Initial kernel scaffold; baseline (speedup 1.0000x reference)
import jax
import jax.numpy as jnp
from jax.experimental import pallas as pl


def kernel(x, edge_index, Wm1, bm1, Wm2, bm2, Wm3, bm3, Wl1, bl1, Wl2, bl2, Wl3, bl3):
    raise NotImplementedError("write your pallas kernel here")


# speedup vs baseline: 7.7238x; 7.7238x over previous
import functools

import jax
import jax.numpy as jnp
from jax import lax
from jax.experimental import pallas as pl
from jax.experimental.pallas import tpu as pltpu
from jax.experimental.pallas import tpu_sc as plsc

_N = 10000
_E = 320000
_NC = 2
_NS = 16
_CHUNK = 128
_CPT = 80
_GRP = 40
_NBUF = 2
_EP = _NC * _NS * _CPT * _CHUNK
_NPAD = 10240
_RPT = _NPAD // _NS
_TILE = 1000
_GRID = _N // _TILE


def _sc_mesh():
    return plsc.VectorSubcoreMesh(
        core_axis_name="c", subcore_axis_name="s",
        num_cores=_NC, num_subcores=_NS)


def _deg_body(didx_h, ones_h, zeros_h, out_h,
              di0, di1, ones_v, acc_sh, *sems):
    cid = lax.axis_index("c")
    sid = lax.axis_index("s")
    wid = sid * _NC + cid
    di = [di0, di1]
    pltpu.sync_copy(ones_h, ones_v)
    pltpu.sync_copy(zeros_h, acc_sh.at[pl.ds(sid * _RPT, _RPT)])
    plsc.subcore_barrier()

    def batch(i, carry):
        descs = []
        for b in range(_NBUF):
            base = (wid * _CPT + i * _NBUF + b) * _CHUNK
            pltpu.sync_copy(didx_h.at[pl.ds(base, _CHUNK)], di[b])
            descs.append(pltpu.async_copy(
                ones_v, acc_sh.at[di[b]], sems[b], add=True))
        for d in descs:
            d.wait()
        return carry

    lax.fori_loop(0, _CPT // _NBUF, batch, 0)
    plsc.subcore_barrier()
    pltpu.sync_copy(acc_sh.at[pl.ds(sid * _RPT, _RPT)],
                    out_h.at[cid, pl.ds(sid * _RPT, _RPT)])


def _deg_pass(didx, ones, zeros16):
    return pl.kernel(
        _deg_body,
        out_type=jax.ShapeDtypeStruct((_NC, _NPAD, 128), jnp.float32),
        mesh=_sc_mesh(),
        scratch_types=[
            pltpu.VMEM((_CHUNK,), jnp.int32),
            pltpu.VMEM((_CHUNK,), jnp.int32),
            pltpu.VMEM((_CHUNK, 128), jnp.float32),
            pltpu.VMEM_SHARED((_NPAD, 128), jnp.float32),
        ] + [pltpu.SemaphoreType.DMA] * _NBUF,
    )(didx, ones, zeros16)


def _pass_body(table_h, sidx_h, didx_h, zeros_h, out_h,
               si0, si1, di0, di1, rows_v, acc_sh, *sems):
    cid = lax.axis_index("c")
    sid = lax.axis_index("s")
    wid = sid * _NC + cid
    si = [si0, si1]
    di = [di0, di1]
    pltpu.sync_copy(zeros_h, acc_sh.at[pl.ds(sid * _RPT, _RPT)])
    plsc.subcore_barrier()

    gsems = sems[:_NBUF]
    ssems = sems[_NBUF:]

    def batch(i, carry):
        gd = []
        for b in range(_NBUF):
            base = (wid * _CPT + i * _NBUF + b) * _CHUNK
            pltpu.sync_copy(sidx_h.at[pl.ds(base, _CHUNK)], si[b])
            pltpu.sync_copy(didx_h.at[pl.ds(base, _CHUNK)], di[b])
            gd.append(pltpu.async_copy(
                table_h.at[si[b]], rows_v.at[b], gsems[b]))
        sd = []
        for b in range(_NBUF):
            gd[b].wait()
            sd.append(pltpu.async_copy(
                rows_v.at[b], acc_sh.at[di[b]], ssems[b], add=True))
        for d in sd:
            d.wait()
        return carry

    lax.fori_loop(0, _CPT // _NBUF, batch, 0)
    plsc.subcore_barrier()
    pltpu.sync_copy(acc_sh.at[pl.ds(sid * _RPT, _RPT)],
                    out_h.at[cid, pl.ds(sid * _RPT, _RPT)])


def _edge_pass(table, sidx, didx, zeros128):
    return pl.kernel(
        _pass_body,
        out_type=jax.ShapeDtypeStruct((_NC, _NPAD, 128), jnp.float32),
        mesh=_sc_mesh(),
        scratch_types=[
            pltpu.VMEM((_CHUNK,), jnp.int32),
            pltpu.VMEM((_CHUNK,), jnp.int32),
            pltpu.VMEM((_CHUNK,), jnp.int32),
            pltpu.VMEM((_CHUNK,), jnp.int32),
            pltpu.VMEM((_NBUF, _CHUNK, 128), jnp.float32),
            pltpu.VMEM_SHARED((_NPAD, 128), jnp.float32),
        ] + [pltpu.SemaphoreType.DMA] * (2 * _NBUF),
    )(table, sidx, didx, zeros128)


def _tc1_body(x_ref, w_ref, d0_ref, d1_ref, dis_ref, u1_ref):
    deg = d0_ref[...][:, 0:1] + d1_ref[...][:, 0:1] + 1.0
    dis = lax.rsqrt(deg)
    dis128 = jnp.broadcast_to(dis, (_TILE, 128))
    hw = jnp.dot(x_ref[...], w_ref[...], preferred_element_type=jnp.float32)
    dis_ref[...] = dis128
    u1_ref[...] = hw * dis128


def _tc1(x, w1cat, d0, d1):
    row = lambda i: (i, 0)
    return pl.pallas_call(
        _tc1_body,
        grid=(_GRID,),
        in_specs=[
            pl.BlockSpec((_TILE, 128), row),
            pl.BlockSpec((128, 128), lambda i: (0, 0)),
            pl.BlockSpec((_TILE, 16), row),
            pl.BlockSpec((_TILE, 16), row),
        ],
        out_specs=[pl.BlockSpec((_TILE, 128), row),
                   pl.BlockSpec((_TILE, 128), row)],
        out_shape=[jax.ShapeDtypeStruct((_N, 128), jnp.float32),
                   jax.ShapeDtypeStruct((_N, 128), jnp.float32)],
    )(x, w1cat, d0, d1)


def _tc2_body(a0_ref, a1_ref, u1_ref, dis_ref, wm_ref, wl_ref, bm_ref, bl_ref,
              u2m_ref, u2l_ref):
    dis = dis_ref[...]
    t = dis * (a0_ref[...] + a1_ref[...] + u1_ref[...])
    hm = jax.nn.relu(t[:, :64] + bm_ref[...])
    hl = jax.nn.relu(t[:, 64:] + bl_ref[...])
    u2m_ref[...] = jnp.dot(hm, wm_ref[...],
                           preferred_element_type=jnp.float32) * dis
    u2l_ref[...] = jnp.dot(hl, wl_ref[...],
                           preferred_element_type=jnp.float32) * dis


def _tc2(a0, a1, u1, dis, wm2, wl2, bm1, bl1):
    row = lambda i: (i, 0)
    full = lambda i: (0, 0)
    return pl.pallas_call(
        _tc2_body,
        grid=(_GRID,),
        in_specs=[
            pl.BlockSpec((_TILE, 128), row),
            pl.BlockSpec((_TILE, 128), row),
            pl.BlockSpec((_TILE, 128), row),
            pl.BlockSpec((_TILE, 128), row),
            pl.BlockSpec((64, 128), full),
            pl.BlockSpec((64, 128), full),
            pl.BlockSpec((1, 64), full),
            pl.BlockSpec((1, 64), full),
        ],
        out_specs=[pl.BlockSpec((_TILE, 128), row),
                   pl.BlockSpec((_TILE, 128), row)],
        out_shape=[jax.ShapeDtypeStruct((_N, 128), jnp.float32),
                   jax.ShapeDtypeStruct((_N, 128), jnp.float32)],
    )(a0, a1, u1, dis, wm2, wl2, bm1, bl1)


def _tc3_body(am0_ref, am1_ref, u2m_ref, al0_ref, al1_ref, u2l_ref, dis_ref,
              wm_ref, wl_ref, bm_ref, bl_ref, u3_ref):
    dis = dis_ref[...]
    tm = jax.nn.relu(dis * (am0_ref[...] + am1_ref[...] + u2m_ref[...])
                     + bm_ref[...])
    tl = jax.nn.relu(dis * (al0_ref[...] + al1_ref[...] + u2l_ref[...])
                     + bl_ref[...])
    u3m = jnp.dot(tm, wm_ref[...], preferred_element_type=jnp.float32)
    u3l = jnp.dot(tl, wl_ref[...], preferred_element_type=jnp.float32)
    u3_ref[...] = jnp.concatenate([u3m, u3l], axis=1) * dis


def _tc3(am0, am1, u2m, al0, al1, u2l, dis, wm3, wl3, bm2, bl2):
    row = lambda i: (i, 0)
    full = lambda i: (0, 0)
    return pl.pallas_call(
        _tc3_body,
        grid=(_GRID,),
        in_specs=[pl.BlockSpec((_TILE, 128), row)] * 6 + [
            pl.BlockSpec((_TILE, 128), row),
            pl.BlockSpec((128, 64), full),
            pl.BlockSpec((128, 64), full),
            pl.BlockSpec((1, 128), full),
            pl.BlockSpec((1, 128), full),
        ],
        out_specs=[pl.BlockSpec((_TILE, 128), row)],
        out_shape=[jax.ShapeDtypeStruct((_N, 128), jnp.float32)],
    )(am0, am1, u2m, al0, al1, u2l, dis, wm3, wl3, bm2, bl2)[0]


def _tc4_body(a0_ref, a1_ref, u3_ref, dis_ref, bm_ref, bl_ref,
              mu_ref, lg_ref):
    t = dis_ref[...] * (a0_ref[...] + a1_ref[...] + u3_ref[...])
    mu_ref[...] = jax.nn.relu(t[:, :64] + bm_ref[...])
    lg_ref[...] = jax.nn.relu(t[:, 64:] + bl_ref[...])


def _tc4(a0, a1, u3, dis, bm3, bl3):
    row = lambda i: (i, 0)
    full = lambda i: (0, 0)
    return pl.pallas_call(
        _tc4_body,
        grid=(_GRID,),
        in_specs=[
            pl.BlockSpec((_TILE, 128), row),
            pl.BlockSpec((_TILE, 128), row),
            pl.BlockSpec((_TILE, 128), row),
            pl.BlockSpec((_TILE, 128), row),
            pl.BlockSpec((1, 64), full),
            pl.BlockSpec((1, 64), full),
        ],
        out_specs=[pl.BlockSpec((_TILE, 64), row),
                   pl.BlockSpec((_TILE, 64), row)],
        out_shape=[jax.ShapeDtypeStruct((_N, 64), jnp.float32),
                   jax.ShapeDtypeStruct((_N, 64), jnp.float32)],
    )(a0, a1, u3, dis, bm3, bl3)


def kernel(x, edge_index, Wm1, bm1, Wm2, bm2, Wm3, bm3,
           Wl1, bl1, Wl2, bl2, Wl3, bl3):
    src = edge_index[0]
    dst = edge_index[1]
    npad = _EP - _E
    pad_i = jnp.arange(npad, dtype=jnp.int32)
    sidx = jnp.concatenate([src, jnp.zeros((npad,), jnp.int32)])
    didx = jnp.concatenate([dst, _N + (pad_i % 128)])

    ones128 = jnp.ones((_CHUNK, 128), jnp.float32)
    zeros128 = jnp.zeros((_RPT, 128), jnp.float32)

    degacc = _deg_pass(didx, ones128, zeros128)
    d0 = degacc[0, :_N, :16]
    d1 = degacc[1, :_N, :16]

    w1cat = jnp.concatenate([Wm1, Wl1], axis=1)
    dis, u1 = _tc1(x, w1cat, d0, d1)

    acc1 = _edge_pass(u1, sidx, didx, zeros128)
    u2m, u2l = _tc2(acc1[0, :_N, :], acc1[1, :_N, :], u1, dis,
                    Wm2, Wl2, bm1.reshape(1, 64), bl1.reshape(1, 64))

    acc2m = _edge_pass(u2m, sidx, didx, zeros128)
    acc2l = _edge_pass(u2l, sidx, didx, zeros128)
    u3 = _tc3(acc2m[0, :_N, :], acc2m[1, :_N, :], u2m,
              acc2l[0, :_N, :], acc2l[1, :_N, :], u2l, dis,
              Wm3, Wl3, bm2.reshape(1, 128), bl2.reshape(1, 128))

    acc3 = _edge_pass(u3, sidx, didx, zeros128)
    mu, log = _tc4(acc3[0, :_N, :], acc3[1, :_N, :], u3, dis,
                   bm3.reshape(1, 64), bl3.reshape(1, 64))
    return (mu, log)
